# Initial kernel scaffold; baseline (speedup 1.0000x reference)
import jax
import jax.numpy as jnp
from jax.experimental import pallas as pl


def kernel(x, W1, g1, b1, W2, g2, b2, W3, g3, b3, W4, g4, b4, W5, g5, b5, W6, g6, b6):
    raise NotImplementedError("write your pallas kernel here")


# speedup vs baseline: 1.8620x; 1.8620x over previous
import functools

import jax
import jax.numpy as jnp
from jax.experimental import pallas as pl

K = 8
EPS = 1e-5
N = 1024
B = 2

_DN = (((0,), (0,)), ((), ()))


def _dot_chunked(xb, chunk=256):
    d = xb.shape[0]
    if d <= chunk:
        return jax.lax.dot_general(xb, xb, _DN, preferred_element_type=jnp.float32)
    acc = None
    c0 = 0
    while c0 < d:
        c1 = min(c0 + chunk, d)
        p = jax.lax.dot_general(xb[c0:c1], xb[c0:c1], _DN,
                                preferred_element_type=jnp.float32)
        acc = p if acc is None else acc + p
        c0 = c1
    return acc


def _knn_kernel(x_ref, idx_ref):
    xb = x_ref[0]
    t = _dot_chunked(xb)
    inner = -2.0 * t
    xx = jnp.sum(xb * xb, axis=0)
    pd = (0.0 - xx)[None, :] - inner
    pd = pd - xx[:, None]
    cols = jax.lax.broadcasted_iota(jnp.int32, (N, N), 1)
    work = pd
    rows = []
    for _ in range(K):
        m = jnp.max(work, axis=1, keepdims=True)
        sel = jnp.where(work == m, cols, N)
        j = jnp.min(sel, axis=1, keepdims=True)
        rows.append(j)
        work = jnp.where(cols == j, -jnp.inf, work)
    idx_ref[0] = jnp.concatenate(rows, axis=1).astype(jnp.int32)


def _knn(x):
    d = x.shape[1]
    f = pl.pallas_call(
        _knn_kernel,
        out_shape=jax.ShapeDtypeStruct((B, N, K), jnp.int32),
        grid=(B,),
        in_specs=[pl.BlockSpec((1, d, N), lambda b: (b, 0, 0))],
        out_specs=pl.BlockSpec((1, N, K), lambda b: (b, 0, 0)),
    )
    return f(x)


def _get_graph_feature(x):
    b, d, n = x.shape
    idx = _knn(x)
    xt = jnp.transpose(x, (0, 2, 1))
    feature = jax.vmap(lambda t, i: t[i])(xt, idx)
    center = jnp.broadcast_to(xt[:, :, None, :], (b, n, K, d))
    feat = jnp.concatenate([feature - center, center], axis=3)
    return jnp.transpose(feat, (0, 3, 1, 2))


def _conv_bn(x, W, g, bt, relu=True):
    y = jnp.einsum('oc,bcnk->bonk', W, x)
    mean = jnp.mean(y, axis=(0, 2, 3), keepdims=True)
    var = jnp.var(y, axis=(0, 2, 3), keepdims=True)
    y = g.reshape(1, -1, 1, 1) * (y - mean) / jnp.sqrt(var + EPS) + bt.reshape(1, -1, 1, 1)
    if relu:
        y = jax.nn.relu(y)
    return y


def kernel(x, W1, g1, b1, W2, g2, b2, W3, g3, b3, W4, g4, b4, W5, g5, b5, W6, g6, b6):
    npoints = x.shape[2]
    h = _get_graph_feature(x)
    h = _conv_bn(h, W1, g1, b1).max(axis=-1)
    pointfeat = h
    h = _get_graph_feature(h)
    h = _conv_bn(h, W2, g2, b2).max(axis=-1)
    h = _get_graph_feature(h)
    h = _conv_bn(h, W3, g3, b3, relu=False).max(axis=-1)
    h = jnp.max(h, axis=2)
    h = jnp.broadcast_to(h[:, :, None], (h.shape[0], 1024, npoints))
    h = jnp.concatenate([h, pointfeat], axis=1)
    h = _get_graph_feature(h)
    h = _conv_bn(h, W4, g4, b4).max(axis=-1)
    h = _get_graph_feature(h)
    h = _conv_bn(h, W5, g5, b5).max(axis=-1)
    h = _get_graph_feature(h)
    h = _conv_bn(h, W6, g6, b6).max(axis=-1)
    return h
